# Initial kernel scaffold; baseline (speedup 1.0000x reference)
import functools

import jax
import jax.numpy as jnp
from jax import lax
from jax.experimental import pallas as pl
from jax.experimental.pallas import tpu as pltpu
from jax.experimental.pallas import tpu_sc as plsc

VOCAB = 100000
HID = 128
MAX_POS = 512
B = 512
S = 512
N = B * S
EPS = 1e-12

NC = 2
NS = 16
NW = NC * NS
TOK_PER_W = N // NW
CHUNK = 128
NCHUNK = TOK_PER_W // CHUNK
LANES = 16
NGROUP = CHUNK // LANES


def _rsqrt(x):
    i = lax.bitcast_convert_type(x, jnp.int32)
    y = lax.bitcast_convert_type(
        jnp.int32(0x5F3759DF) - lax.shift_right_logical(i, 1), jnp.float32)
    for _ in range(3):
        y = y * (1.5 - 0.5 * x * y * y)
    return y


def _sc_body(ids_hbm, tt_hbm, word_hbm, posT_hbm, dty_hbm, gam_hbm, bet_hbm,
             out_hbm, idx_v, tt_v, x_v, xT_v, posT_v, dty_v, gam_v, bet_v,
             sem):
    wid = lax.axis_index("s") * NC + lax.axis_index("c")
    base_w = wid * TOK_PER_W

    pltpu.sync_copy(posT_hbm, posT_v)
    pltpu.sync_copy(dty_hbm, dty_v)
    pltpu.sync_copy(gam_hbm, gam_v)
    pltpu.sync_copy(bet_hbm, bet_v)

    zeros = jnp.zeros((LANES,), jnp.float32)
    iota = lax.iota(jnp.int32, LANES)

    def chunk_body(k, carry):
        cbase = base_w + k * CHUNK
        pbase = lax.rem(cbase, S)
        pltpu.sync_copy(ids_hbm.at[pl.ds(cbase, CHUNK)], idx_v)
        pltpu.sync_copy(tt_hbm.at[pl.ds(cbase, CHUNK)], tt_v)
        pltpu.async_copy(word_hbm.at[idx_v], x_v, sem).wait()

        def group_body(g, carry2):
            t0 = g * LANES
            ttf = tt_v[pl.ds(t0, LANES)].astype(jnp.float32)
            tvec = t0 + iota

            def col1(c, carry3):
                acc, accsq = carry3
                cc = jnp.full((LANES,), c, jnp.int32)
                g16 = plsc.load_gather(x_v, [tvec, cc])
                p = posT_v[c, pl.ds(pbase + t0, LANES)]
                x = g16 + p + ttf * dty_v[c, :]
                xT_v[c, :] = x
                return (acc + x, accsq + x * x)

            acc, accsq = lax.fori_loop(0, HID, col1, (zeros, zeros))
            mean = acc * (1.0 / HID)
            var = accsq * (1.0 / HID) - mean * mean
            a_v = _rsqrt(var + EPS)
            b_v = -mean * a_v

            def col2(c, carry3):
                cc = jnp.full((LANES,), c, jnp.int32)
                y = (xT_v[c, :] * a_v + b_v) * gam_v[c, :] + bet_v[c, :]
                plsc.store_scatter(x_v, [tvec, cc], y)
                return carry3

            lax.fori_loop(0, HID, col2, 0)
            return carry2

        lax.fori_loop(0, NGROUP, group_body, 0)
        pltpu.sync_copy(x_v, out_hbm.at[pl.ds(cbase, CHUNK)])
        return carry

    lax.fori_loop(0, NCHUNK, chunk_body, 0)


@jax.jit
def kernel(input_ids, token_type_ids, word_emb, pos_emb, type_emb, ln_gamma,
           ln_beta):
    ids = input_ids.reshape(-1).astype(jnp.int32)
    tts = token_type_ids.reshape(-1).astype(jnp.int32)
    posT = pos_emb.astype(jnp.float32).T + type_emb[0][:, None]
    dty = jnp.broadcast_to((type_emb[1] - type_emb[0])[:, None], (HID, LANES))
    gam = jnp.broadcast_to(ln_gamma[:, None], (HID, LANES))
    bet = jnp.broadcast_to(ln_beta[:, None], (HID, LANES))

    run = pl.kernel(
        _sc_body,
        out_type=jax.ShapeDtypeStruct((N, HID), jnp.float32),
        mesh=plsc.VectorSubcoreMesh(core_axis_name="c", subcore_axis_name="s"),
        scratch_types=[
            pltpu.VMEM((CHUNK,), jnp.int32),
            pltpu.VMEM((CHUNK,), jnp.int32),
            pltpu.VMEM((CHUNK, HID), jnp.float32),
            pltpu.VMEM((HID, LANES), jnp.float32),
            pltpu.VMEM((HID, S), jnp.float32),
            pltpu.VMEM((HID, LANES), jnp.float32),
            pltpu.VMEM((HID, LANES), jnp.float32),
            pltpu.VMEM((HID, LANES), jnp.float32),
            pltpu.SemaphoreType.DMA,
        ],
    )
    out = run(ids, tts, word_emb.astype(jnp.float32), posT,
              jnp.ascontiguousarray(dty), jnp.ascontiguousarray(gam),
              jnp.ascontiguousarray(bet))
    return out.reshape(B, S, HID)


# speedup vs baseline: 1.3494x; 1.3494x over previous
import functools

import jax
import jax.numpy as jnp
from jax import lax
from jax.experimental import pallas as pl
from jax.experimental.pallas import tpu as pltpu
from jax.experimental.pallas import tpu_sc as plsc

VOCAB = 100000
HID = 128
MAX_POS = 512
B = 512
S = 512
N = B * S
EPS = 1e-12

NC = 2
NS = 16
NW = NC * NS
TOK_PER_W = N // NW
CHUNK = 128
NCHUNK = TOK_PER_W // CHUNK
LANES = 16
NGROUP = CHUNK // LANES


def _rsqrt(x):
    i = lax.bitcast_convert_type(x, jnp.int32)
    y = lax.bitcast_convert_type(
        jnp.int32(0x5F3759DF) - lax.shift_right_logical(i, 1), jnp.float32)
    for _ in range(3):
        y = y * (1.5 - 0.5 * x * y * y)
    return y


def _sc_body(ids_hbm, tt_hbm, word_hbm, posT_hbm, dty_hbm, gam_hbm, bet_hbm,
             out_hbm, idx_v, tt_v, x_v, xT_v, posT_v, dty_v, gam_v, bet_v,
             sem):
    wid = lax.axis_index("s") * NC + lax.axis_index("c")
    base_w = wid * TOK_PER_W

    pltpu.sync_copy(dty_hbm, dty_v)
    pltpu.sync_copy(gam_hbm, gam_v)
    pltpu.sync_copy(bet_hbm, bet_v)

    zeros = jnp.zeros((LANES,), jnp.float32)
    iota = lax.iota(jnp.int32, LANES)

    def chunk_body(k, h, carry):
        kk = 4 * (k // 2) + 2 * h + lax.rem(k, 2)
        cbase = base_w + kk * CHUNK
        pbase = lax.rem(cbase, S) - h * (S // 2)
        pltpu.sync_copy(ids_hbm.at[pl.ds(cbase, CHUNK)], idx_v)
        pltpu.sync_copy(tt_hbm.at[pl.ds(cbase, CHUNK)], tt_v)
        pltpu.async_copy(word_hbm.at[idx_v], x_v, sem).wait()

        def group_body(g, carry2):
            t0 = g * LANES
            ttf = tt_v[pl.ds(t0, LANES)].astype(jnp.float32)
            tvec = t0 + iota

            def col1(c, carry3):
                acc, accsq = carry3
                cc = jnp.full((LANES,), c, jnp.int32)
                g16 = plsc.load_gather(x_v, [tvec, cc])
                p = posT_v[c, pl.ds(pbase + t0, LANES)]
                x = g16 + p + ttf * dty_v[c, :]
                xT_v[c, :] = x
                return (acc + x, accsq + x * x)

            acc, accsq = lax.fori_loop(0, HID, col1, (zeros, zeros))
            mean = acc * (1.0 / HID)
            var = accsq * (1.0 / HID) - mean * mean
            a_v = _rsqrt(var + EPS)
            b_v = -mean * a_v

            def col2(c, carry3):
                cc = jnp.full((LANES,), c, jnp.int32)
                y = (xT_v[c, :] * a_v + b_v) * gam_v[c, :] + bet_v[c, :]
                plsc.store_scatter(x_v, [tvec, cc], y)
                return carry3

            lax.fori_loop(0, HID, col2, 0)
            return carry2

        lax.fori_loop(0, NGROUP, group_body, 0)
        pltpu.sync_copy(x_v, out_hbm.at[pl.ds(cbase, CHUNK)])
        return carry

    for h in range(2):
        pltpu.sync_copy(posT_hbm.at[:, pl.ds(h * (S // 2), S // 2)], posT_v)
        lax.fori_loop(0, NCHUNK // 2,
                      lambda k, c, _h=h: chunk_body(k, _h, c), 0)


@jax.jit
def kernel(input_ids, token_type_ids, word_emb, pos_emb, type_emb, ln_gamma,
           ln_beta):
    ids = input_ids.reshape(-1).astype(jnp.int32)
    tts = token_type_ids.reshape(-1).astype(jnp.int32)
    posT = pos_emb.astype(jnp.float32).T + type_emb[0][:, None]
    dty = jnp.broadcast_to((type_emb[1] - type_emb[0])[:, None], (HID, LANES))
    gam = jnp.broadcast_to(ln_gamma[:, None], (HID, LANES))
    bet = jnp.broadcast_to(ln_beta[:, None], (HID, LANES))

    run = pl.kernel(
        _sc_body,
        out_type=jax.ShapeDtypeStruct((N, HID), jnp.float32),
        mesh=plsc.VectorSubcoreMesh(core_axis_name="c", subcore_axis_name="s",
                                    num_cores=NC, num_subcores=NS),
        compiler_params=pltpu.CompilerParams(needs_layout_passes=False),
        scratch_types=[
            pltpu.VMEM((CHUNK,), jnp.int32),
            pltpu.VMEM((CHUNK,), jnp.int32),
            pltpu.VMEM((CHUNK, HID), jnp.float32),
            pltpu.VMEM((HID, LANES), jnp.float32),
            pltpu.VMEM((HID, S // 2), jnp.float32),
            pltpu.VMEM((HID, LANES), jnp.float32),
            pltpu.VMEM((HID, LANES), jnp.float32),
            pltpu.VMEM((HID, LANES), jnp.float32),
            pltpu.SemaphoreType.DMA,
        ],
    )
    out = run(ids, tts, word_emb.astype(jnp.float32), posT, dty, gam, bet)
    return out.reshape(B, S, HID)
